# baseline (speedup 1.0000x reference)
import functools

import jax
import jax.numpy as jnp
from jax import lax
from jax.experimental import pallas as pl
from jax.experimental.pallas import tpu as pltpu
from jax.experimental.pallas import tpu_sc as plsc

LANES = 16
CHUNKS = (128, 128, 128, 64, 32, 16, 16)
DEPTH = 2
ROWS_PER_ITER = 8


def _rsqrt_newton(x):
    i = lax.bitcast_convert_type(x, jnp.int32)
    i = jnp.int32(0x5F3759DF) - (i >> 1)
    y = lax.bitcast_convert_type(i, jnp.float32)
    half_x = x * 0.5
    for _ in range(2):
        y = y * (1.5 - half_x * y * y)
    return y


def _hsum_splat(v):
    lanes = lax.iota(jnp.int32, LANES)
    for k in (1, 2, 4, 8):
        shuf = lanes ^ k
        v = v + v.at[shuf].get(mode="promise_in_bounds")
    return v


def _make_kernel(V, D, B):
    info = plsc.get_sparse_core_info()
    nc, ns = info.num_cores, info.num_subcores
    nw = nc * ns
    assert B % nw == 0
    b_per_w = B // nw
    assert sum(CHUNKS) == b_per_w
    offs = [sum(CHUNKS[:c]) for c in range(len(CHUNKS))]
    n_chunks = len(CHUNKS)
    vecs = D // LANES
    mesh = plsc.VectorSubcoreMesh(core_axis_name="c", subcore_axis_name="s")

    @functools.partial(
        pl.kernel,
        mesh=mesh,
        out_type=jax.ShapeDtypeStruct((B, D), jnp.float32),
        scratch_types=[
            pltpu.VMEM((b_per_w,), jnp.int32),
            pltpu.VMEM((b_per_w, D), jnp.float32),
        ]
        + [pltpu.SemaphoreType.DMA] * n_chunks
        + [pltpu.SemaphoreType.DMA],
    )
    def k(table_hbm, nodes_hbm, out_hbm, idx_v, rows_v, *sems):
        gather_sems, out_sem = sems[:n_chunks], sems[n_chunks]
        wid = lax.axis_index("s") * nc + lax.axis_index("c")
        base = wid * b_per_w
        pltpu.sync_copy(nodes_hbm.at[pl.ds(base, b_per_w)], idx_v)

        def fire(c):
            return pltpu.async_copy(
                table_hbm.at[idx_v.at[pl.ds(offs[c], CHUNKS[c])]],
                rows_v.at[pl.ds(offs[c], CHUNKS[c])],
                gather_sems[c],
            )

        gathers = {c: fire(c) for c in range(min(DEPTH, n_chunks))}

        def body(it, carry):
            r0 = it * ROWS_PER_ITER
            for k_ in range(ROWS_PER_ITER):
                r = r0 + k_
                vs = [rows_v[r, pl.ds(j * LANES, LANES)] for j in range(vecs)]
                sq = [v * v for v in vs]
                while len(sq) > 1:
                    sq = [sq[i] + sq[i + 1] for i in range(0, len(sq) - 1, 2)] \
                        + ([sq[-1]] if len(sq) % 2 else [])
                s = _hsum_splat(sq[0])
                inv = _rsqrt_newton(jnp.maximum(s, 1e-24))
                for j in range(vecs):
                    rows_v[r, pl.ds(j * LANES, LANES)] = vs[j] * inv
            return carry

        writes = []
        for c in range(n_chunks):
            gathers[c].wait()
            if c + DEPTH < n_chunks:
                gathers[c + DEPTH] = fire(c + DEPTH)
            lax.fori_loop(offs[c] // ROWS_PER_ITER,
                          (offs[c] + CHUNKS[c]) // ROWS_PER_ITER, body, 0)
            writes.append(pltpu.async_copy(
                rows_v.at[pl.ds(offs[c], CHUNKS[c])],
                out_hbm.at[pl.ds(base + offs[c], CHUNKS[c])],
                out_sem,
            ))
        for w in writes:
            w.wait()

    return k


@jax.jit
def kernel(table, nodes):
    V, D = table.shape
    B = nodes.shape[0]
    k = _make_kernel(V, D, B)
    return k(table, nodes.astype(jnp.int32))


# speedup vs baseline: 1.0379x; 1.0379x over previous
import functools

import jax
import jax.numpy as jnp
from jax import lax
from jax.experimental import pallas as pl
from jax.experimental.pallas import tpu as pltpu
from jax.experimental.pallas import tpu_sc as plsc

LANES = 16
CHUNKS = (128, 128, 128, 128)
DEPTH = 2
ROWS_PER_ITER = 8


def _rsqrt_newton(x):
    i = lax.bitcast_convert_type(x, jnp.int32)
    i = jnp.int32(0x5F3759DF) - (i >> 1)
    y = lax.bitcast_convert_type(i, jnp.float32)
    half_x = x * 0.5
    for _ in range(2):
        y = y * (1.5 - half_x * y * y)
    return y


def _hsum_splat(v):
    lanes = lax.iota(jnp.int32, LANES)
    for k in (1, 2, 4, 8):
        shuf = lanes ^ k
        v = v + v.at[shuf].get(mode="promise_in_bounds")
    return v


def _make_kernel(V, D, B):
    info = plsc.get_sparse_core_info()
    nc, ns = info.num_cores, info.num_subcores
    nw = nc * ns
    assert B % nw == 0
    b_per_w = B // nw
    assert sum(CHUNKS) == b_per_w
    offs = [sum(CHUNKS[:c]) for c in range(len(CHUNKS))]
    n_chunks = len(CHUNKS)
    vecs = D // LANES
    mesh = plsc.VectorSubcoreMesh(core_axis_name="c", subcore_axis_name="s")

    @functools.partial(
        pl.kernel,
        mesh=mesh,
        out_type=jax.ShapeDtypeStruct((B, D), jnp.float32),
        scratch_types=[
            pltpu.VMEM((b_per_w,), jnp.int32),
            pltpu.VMEM((b_per_w, D), jnp.float32),
        ]
        + [pltpu.SemaphoreType.DMA] * n_chunks
        + [pltpu.SemaphoreType.DMA],
    )
    def k(table_hbm, nodes_hbm, out_hbm, idx_v, rows_v, *sems):
        gather_sems, out_sem = sems[:n_chunks], sems[n_chunks]
        wid = lax.axis_index("s") * nc + lax.axis_index("c")
        base = wid * b_per_w
        pltpu.sync_copy(nodes_hbm.at[pl.ds(base, b_per_w)], idx_v)

        def fire(c):
            return pltpu.async_copy(
                table_hbm.at[idx_v.at[pl.ds(offs[c], CHUNKS[c])]],
                rows_v.at[pl.ds(offs[c], CHUNKS[c])],
                gather_sems[c],
            )

        gathers = {c: fire(c) for c in range(min(DEPTH, n_chunks))}

        def body(it, carry):
            r0 = it * ROWS_PER_ITER
            for k_ in range(ROWS_PER_ITER):
                r = r0 + k_
                vs = [rows_v[r, pl.ds(j * LANES, LANES)] for j in range(vecs)]
                sq = [v * v for v in vs]
                while len(sq) > 1:
                    sq = [sq[i] + sq[i + 1] for i in range(0, len(sq) - 1, 2)] \
                        + ([sq[-1]] if len(sq) % 2 else [])
                s = _hsum_splat(sq[0])
                inv = _rsqrt_newton(jnp.maximum(s, 1e-24))
                for j in range(vecs):
                    rows_v[r, pl.ds(j * LANES, LANES)] = vs[j] * inv
            return carry

        writes = []
        for c in range(n_chunks):
            gathers[c].wait()
            if c + DEPTH < n_chunks:
                gathers[c + DEPTH] = fire(c + DEPTH)
            lax.fori_loop(offs[c] // ROWS_PER_ITER,
                          (offs[c] + CHUNKS[c]) // ROWS_PER_ITER, body, 0)
            writes.append(pltpu.async_copy(
                rows_v.at[pl.ds(offs[c], CHUNKS[c])],
                out_hbm.at[pl.ds(base + offs[c], CHUNKS[c])],
                out_sem,
            ))
        for w in writes:
            w.wait()

    return k


@jax.jit
def kernel(table, nodes):
    V, D = table.shape
    B = nodes.shape[0]
    k = _make_kernel(V, D, B)
    return k(table, nodes.astype(jnp.int32))
